# baseline (speedup 1.0000x reference)
import functools

import jax
import jax.numpy as jnp
from jax import lax
from jax.experimental import pallas as pl
from jax.experimental.pallas import tpu as pltpu
from jax.experimental.pallas import tpu_sc as plsc

EMB = 64
SUBT = 2
BW = 128
CH = SUBT * BW
_NC = 2
_NS = 16
_NW = _NC * _NS


@functools.lru_cache(maxsize=None)
def _make_gather(NB: int, SEQ: int, V2: int):
    PS = SEQ + (-SEQ) % 8
    n_win = PS // 8
    subs = [
        (w, s)
        for w in range(n_win)
        for s in range(8 // SUBT)
        if w * 8 + s * SUBT < SEQ
    ]

    mesh = plsc.VectorSubcoreMesh(core_axis_name="c", subcore_axis_name="s")

    @functools.partial(
        pl.kernel,
        mesh=mesh,
        out_type=jax.ShapeDtypeStruct((SEQ, EMB, NB), jnp.float32),
        scratch_types=[
            pltpu.VMEM((8, BW), jnp.int32),
            pltpu.VMEM((8, BW), jnp.int32),
            pltpu.VMEM((CH,), jnp.int32),
            pltpu.VMEM((CH,), jnp.int32),
            pltpu.VMEM((CH, 128), jnp.float32),
            pltpu.VMEM((CH, 128), jnp.float32),
            pltpu.VMEM((EMB, BW), jnp.float32),
            pltpu.VMEM((EMB, BW), jnp.float32),
            pltpu.VMEM((EMB, BW), jnp.float32),
            pltpu.VMEM((EMB, BW), jnp.float32),
            pltpu.SemaphoreType.DMA,
            pltpu.SemaphoreType.DMA,
        ],
        compiler_params=pltpu.CompilerParams(needs_layout_passes=False),
    )
    def k(
        ids_hbm,
        tab_hbm,
        out_hbm,
        it0,
        it1,
        ix0,
        ix1,
        r0,
        r1,
        tb00,
        tb01,
        tb10,
        tb11,
        gsem,
        osem,
    ):
        wid = lax.axis_index("s") * _NC + lax.axis_index("c")
        bc = wid * BW
        its = (it0, it1)
        ixs = (ix0, ix1)
        rws = (r0, r1)
        tbs = ((tb00, tb01), (tb10, tb11))
        iota = lax.iota(jnp.int32, 16)

        def stage(w):
            pltpu.sync_copy(
                ids_hbm.at[pl.ds(w * 8, 8), pl.ds(bc, BW)], its[w % 2]
            )

        def build_idx(n):
            w, s = subs[n]
            tile = its[w % 2]
            ib = ixs[n % 2]
            for tl in range(SUBT):
                for g in range(BW // 16):
                    v = tile[s * SUBT + tl, pl.ds(g * 16, 16)]
                    ib[pl.ds(tl * BW + g * 16, 16)] = v >> 1

        def start_gather(n):
            pltpu.async_copy(tab_hbm.at[ixs[n % 2]], rws[n % 2], gsem)

        def transpose_write(n):
            w, s = subs[n]
            tile = its[w % 2]
            rows = rws[n % 2]
            tbpair = tbs[n % 2]
            rvecs = []
            cbases = []
            for tl in range(SUBT):
                for g in range(BW // 16):
                    idsv = tile[s * SUBT + tl, pl.ds(g * 16, 16)]
                    cbases.append((idsv & 1) * EMB)
                    rvecs.append(tl * BW + g * 16 + iota)

            @pl.loop(0, EMB)
            def _(e):
                for tl in range(SUBT):
                    for g in range(BW // 16):
                        i = tl * (BW // 16) + g
                        vals = plsc.load_gather(
                            rows, [rvecs[i], cbases[i] + e]
                        )
                        tbpair[tl][e, pl.ds(g * 16, 16)] = vals

            for tl in range(SUBT):
                tglob = w * 8 + s * SUBT + tl
                pltpu.async_copy(
                    tbpair[tl], out_hbm.at[tglob, :, pl.ds(bc, BW)], osem
                )

        def wait_out(n):
            w, s = subs[n]
            for tl in range(SUBT):
                tglob = w * 8 + s * SUBT + tl
                pltpu.make_async_copy(
                    tbs[n % 2][tl],
                    out_hbm.at[tglob, :, pl.ds(bc, BW)],
                    osem,
                ).wait()

        stage(0)
        build_idx(0)
        start_gather(0)
        for n in range(len(subs)):
            if n + 1 < len(subs):
                wn, sn = subs[n + 1]
                if sn == 0:
                    stage(wn)
                build_idx(n + 1)
                if n >= 1:
                    wait_out(n - 1)
                start_gather(n + 1)
            pltpu.make_async_copy(
                tab_hbm.at[ixs[n % 2]], rws[n % 2], gsem
            ).wait()
            transpose_write(n)
        wait_out(len(subs) - 2)
        wait_out(len(subs) - 1)

    return k


def kernel(input_ids, table):
    NB, SEQ = input_ids.shape
    V, D = table.shape
    ids_t = input_ids.T.astype(jnp.int32)
    ids_p = jnp.pad(ids_t, ((0, (-SEQ) % 8), (0, 0)))
    tab2 = table.reshape(V * D // 128, 128)
    x = _make_gather(NB, SEQ, V * D // 128)(ids_p, tab2)
    return jnp.transpose(x, (2, 0, 1))


# speedup vs baseline: 1.6773x; 1.6773x over previous
import functools

import jax
import jax.numpy as jnp
from jax import lax
from jax.experimental import pallas as pl
from jax.experimental.pallas import tpu as pltpu
from jax.experimental.pallas import tpu_sc as plsc

EMB = 64
_NC = 2
_NS = 16
_NW = _NC * _NS


@functools.lru_cache(maxsize=None)
def _make_gather(B: int, n_chunks: int, chunk: int):
    b_per_w = B // _NW
    assert b_per_w == n_chunks * chunk

    mesh = plsc.VectorSubcoreMesh(core_axis_name="c", subcore_axis_name="s")

    @functools.partial(
        pl.kernel,
        mesh=mesh,
        out_type=jax.ShapeDtypeStruct((B, EMB), jnp.float32),
        scratch_types=[
            pltpu.VMEM((chunk,), jnp.int32),
            pltpu.VMEM((chunk,), jnp.int32),
            pltpu.VMEM((chunk, EMB), jnp.float32),
            pltpu.VMEM((chunk, EMB), jnp.float32),
            pltpu.SemaphoreType.DMA,
            pltpu.SemaphoreType.DMA,
        ],
        compiler_params=pltpu.CompilerParams(use_tc_tiling_on_sc=False),
    )
    def k(idx_hbm, table_hbm, out_hbm, idx0, idx1, rows0, rows1, gsem, osem):
        wid = lax.axis_index("s") * _NC + lax.axis_index("c")
        base = wid * b_per_w
        idx_v = (idx0, idx1)
        rows_v = (rows0, rows1)

        def idx_src(j):
            return idx_hbm.at[pl.ds(base + j * chunk, chunk)]

        def out_dst(j):
            return out_hbm.at[pl.ds(base + j * chunk, chunk)]

        pltpu.sync_copy(idx_src(0), idx0)
        pltpu.async_copy(table_hbm.at[idx0], rows0, gsem)
        for j in range(n_chunks):
            cur, nxt = j % 2, (j + 1) % 2
            if j + 1 < n_chunks:
                pltpu.sync_copy(idx_src(j + 1), idx_v[nxt])
                if j >= 1:
                    pltpu.make_async_copy(
                        rows_v[nxt], out_dst(j - 1), osem
                    ).wait()
                pltpu.async_copy(table_hbm.at[idx_v[nxt]], rows_v[nxt], gsem)
            pltpu.make_async_copy(
                table_hbm.at[idx_v[cur]], rows_v[cur], gsem
            ).wait()
            pltpu.async_copy(rows_v[cur], out_dst(j), osem)
        for j in (n_chunks - 2, n_chunks - 1):
            pltpu.make_async_copy(rows_v[j % 2], out_dst(j), osem).wait()

    return k


def kernel(input_ids, table):
    B = input_ids.shape[0] * input_ids.shape[1]
    ids_flat = input_ids.reshape(-1).astype(jnp.int32)
    out = _make_gather(B, 8, B // _NW // 8)(ids_flat, table)
    return out.reshape(input_ids.shape + (EMB,))
